# baseline (speedup 1.0000x reference)
import functools

import jax
import jax.numpy as jnp
from jax import lax
from jax.experimental import pallas as pl
from jax.experimental.pallas import tpu as pltpu
from jax.experimental.pallas import tpu_sc as plsc

KTAB = 8192
BATCH = 4096
D = 8192
NC, NS, L = 2, 16, 16
NW = NC * NS
BPW = BATCH // NW
CH = 8
NCHUNK = BPW // CH
UNROLL = 8

_MESH = plsc.VectorSubcoreMesh(core_axis_name="c", subcore_axis_name="s")


def _wid():
    return lax.axis_index("s") * NC + lax.axis_index("c")


def _all_sum(v):
    for sh in (1, 2, 4, 8):
        idx = (lax.iota(jnp.int32, L) + sh) & (L - 1)
        v = v + v.at[idx].get(mode="promise_in_bounds")
    return v


@functools.partial(
    pl.kernel,
    out_type=jax.ShapeDtypeStruct((4, 128), jnp.float32),
    mesh=_MESH,
    scratch_types=[
        pltpu.VMEM((NCHUNK, CH), jnp.int32),
        pltpu.VMEM((CH, D), jnp.float32),
        pltpu.VMEM((L,), jnp.float32),
        pltpu.SemaphoreType.DMA,
    ],
)
def _sum_kernel(probs_hbm, x_hbm, u_hbm, part_hbm, idx_v, buf_v, vec_v, sem):
    wid = _wid()
    base = wid * BPW
    for cc in range(NCHUNK):
        pltpu.sync_copy(x_hbm.at[pl.ds(base + cc * CH, CH)], idx_v.at[cc])
    pltpu.sync_copy(u_hbm, vec_v)
    s_u = _all_sum(vec_v[...])
    a = 1.0 - s_u
    b = s_u * (1.0 / KTAB)

    def chunk_body(c, accs):
        pltpu.async_copy(
            probs_hbm.at[idx_v.at[c]], buf_v, sem
        ).wait()

        def make_row(r):
            def body(j, accs):
                out = []
                for k in range(UNROLL):
                    v = buf_v[r, pl.ds((j * UNROLL + k) * L, L)]
                    z = jnp.minimum(jnp.maximum(a * v + b, 1e-6), 1.0)
                    out.append(accs[k] + z)
                return tuple(out)
            return body

        for r in range(CH):
            accs = lax.fori_loop(0, D // (L * UNROLL), make_row(r), accs)
        return accs

    zero = jnp.zeros((L,), jnp.float32)
    accs = lax.fori_loop(0, NCHUNK, chunk_body, (zero,) * UNROLL)
    acc = accs[0]
    for k in range(1, UNROLL):
        acc = acc + accs[k]
    vec_v[...] = acc
    pltpu.sync_copy(vec_v, part_hbm.at[wid // 8, pl.ds((wid % 8) * L, L)])


@functools.partial(
    pl.kernel,
    out_type=jax.ShapeDtypeStruct((BATCH, D), jnp.float32),
    mesh=_MESH,
    scratch_types=[
        pltpu.VMEM((NCHUNK, CH), jnp.int32),
        pltpu.VMEM((CH, D), jnp.float32),
        pltpu.VMEM((L,), jnp.float32),
        pltpu.VMEM((4, 128), jnp.float32),
        pltpu.SemaphoreType.DMA,
    ],
)
def _scale_kernel(probs_hbm, x_hbm, u_hbm, part_hbm, out_hbm,
                  idx_v, buf_v, vec_v, part_v, sem):
    wid = _wid()
    base = wid * BPW
    for cc in range(NCHUNK):
        pltpu.sync_copy(x_hbm.at[pl.ds(base + cc * CH, CH)], idx_v.at[cc])
    pltpu.sync_copy(u_hbm, vec_v)
    pltpu.sync_copy(part_hbm, part_v)
    s_u = _all_sum(vec_v[...])

    tot = jnp.zeros((L,), jnp.float32)
    for r in range(4):
        for j in range(8):
            tot = tot + part_v[r, pl.ds(j * L, L)]
    r_s = 1.0 / _all_sum(tot)

    a2 = (1.0 - s_u) * r_s
    b2 = (s_u * (1.0 / KTAB)) * r_s
    lo = 1e-6 * r_s
    hi = r_s

    def chunk_body(c, carry):
        pltpu.async_copy(
            probs_hbm.at[idx_v.at[c]], buf_v, sem
        ).wait()

        def make_row(r):
            def body(j, carry):
                for k in range(UNROLL):
                    col = (j * UNROLL + k) * L
                    v = buf_v[r, pl.ds(col, L)]
                    buf_v[r, pl.ds(col, L)] = jnp.minimum(
                        jnp.maximum(a2 * v + b2, lo), hi)
                return carry
            return body

        for r in range(CH):
            lax.fori_loop(0, D // (L * UNROLL), make_row(r), 0)
        pltpu.sync_copy(buf_v, out_hbm.at[pl.ds(base + c * CH, CH)])
        return carry

    lax.fori_loop(0, NCHUNK, chunk_body, 0)


def kernel(probs, x, u, t_now, t_next):
    x32 = x.astype(jnp.int32)
    part = _sum_kernel(probs, x32, u)
    return _scale_kernel(probs, x32, u, part)


# speedup vs baseline: 1.6509x; 1.6509x over previous
import functools

import jax
import jax.numpy as jnp
from jax import lax
from jax.experimental import pallas as pl
from jax.experimental.pallas import tpu as pltpu
from jax.experimental.pallas import tpu_sc as plsc

KTAB = 8192
BATCH = 4096
D = 8192
NC, NS, L = 2, 16, 16
NW = NC * NS
BPW = BATCH // NW
CH = 2
NCHUNK = BPW // CH
NBUF = 4
UNROLL = 8

_MESH = plsc.VectorSubcoreMesh(core_axis_name="c", subcore_axis_name="s")


def _wid():
    return lax.axis_index("s") * NC + lax.axis_index("c")


def _all_sum(v):
    for sh in (1, 2, 4, 8):
        idx = (lax.iota(jnp.int32, L) + sh) & (L - 1)
        v = v + v.at[idx].get(mode="promise_in_bounds")
    return v


def _gather(probs_hbm, idx_v, c, buf, sem):
    return pltpu.make_async_copy(probs_hbm.at[idx_v.at[c]], buf, sem)


_SCRATCH = [
    pltpu.VMEM((NCHUNK, CH), jnp.int32),
    pltpu.VMEM((CH, D), jnp.float32),
    pltpu.VMEM((CH, D), jnp.float32),
    pltpu.VMEM((CH, D), jnp.float32),
    pltpu.VMEM((CH, D), jnp.float32),
    pltpu.VMEM((L,), jnp.float32),
    pltpu.SemaphoreType.DMA,
    pltpu.SemaphoreType.DMA,
    pltpu.SemaphoreType.DMA,
    pltpu.SemaphoreType.DMA,
]


@functools.partial(
    pl.kernel,
    out_type=jax.ShapeDtypeStruct((4, 128), jnp.float32),
    mesh=_MESH,
    scratch_types=_SCRATCH,
)
def _sum_kernel(probs_hbm, x3_hbm, u_hbm, part_hbm,
                idx_v, b0, b1, b2, b3, vec_v, g0, g1, g2, g3):
    bufs = (b0, b1, b2, b3)
    gsem = (g0, g1, g2, g3)
    wid = _wid()
    pltpu.sync_copy(x3_hbm.at[wid], idx_v)
    pltpu.sync_copy(u_hbm, vec_v)
    s_u = _all_sum(vec_v[...])
    a = 1.0 - s_u
    b = s_u * (1.0 / KTAB)

    for c in range(NBUF - 1):
        _gather(probs_hbm, idx_v, c, bufs[c], gsem[c]).start()

    def step(k, accs):
        for j in range(NBUF):
            c = NBUF * k + j
            _gather(probs_hbm, idx_v, c, bufs[j], gsem[j]).wait()

            def make_row(r, j=j):
                def body(jj, accs):
                    out = []
                    for q in range(UNROLL):
                        v = bufs[j][r, pl.ds((jj * UNROLL + q) * L, L)]
                        z = jnp.minimum(jnp.maximum(a * v + b, 1e-6), 1.0)
                        out.append(accs[q] + z)
                    return tuple(out)
                return body

            for r in range(CH):
                accs = lax.fori_loop(0, D // (L * UNROLL), make_row(r), accs)

            nc = c + NBUF - 1
            nj = (j + NBUF - 1) % NBUF

            @pl.when(nc < NCHUNK)
            def _():
                _gather(probs_hbm, idx_v, nc, bufs[nj], gsem[nj]).start()
        return accs

    zero = jnp.zeros((L,), jnp.float32)
    accs = lax.fori_loop(0, NCHUNK // NBUF, step, (zero,) * UNROLL)
    acc = accs[0]
    for q in range(1, UNROLL):
        acc = acc + accs[q]
    vec_v[...] = acc
    pltpu.sync_copy(vec_v, part_hbm.at[wid // 8, pl.ds((wid % 8) * L, L)])


@functools.partial(
    pl.kernel,
    out_type=jax.ShapeDtypeStruct((BATCH, D), jnp.float32),
    mesh=_MESH,
    scratch_types=_SCRATCH + [
        pltpu.VMEM((4, 128), jnp.float32),
        pltpu.SemaphoreType.DMA,
        pltpu.SemaphoreType.DMA,
        pltpu.SemaphoreType.DMA,
        pltpu.SemaphoreType.DMA,
    ],
)
def _scale_kernel(probs_hbm, x3_hbm, u_hbm, part_hbm, out_hbm,
                  idx_v, b0, b1, b2, b3, vec_v, g0, g1, g2, g3,
                  part_v, w0, w1, w2, w3):
    bufs = (b0, b1, b2, b3)
    gsem = (g0, g1, g2, g3)
    wsem = (w0, w1, w2, w3)
    wid = _wid()
    base = wid * BPW
    pltpu.sync_copy(x3_hbm.at[wid], idx_v)
    pltpu.sync_copy(u_hbm, vec_v)
    pltpu.sync_copy(part_hbm, part_v)
    s_u = _all_sum(vec_v[...])

    tot = jnp.zeros((L,), jnp.float32)
    for r in range(4):
        for jj in range(8):
            tot = tot + part_v[r, pl.ds(jj * L, L)]
    r_s = 1.0 / _all_sum(tot)

    a2 = (1.0 - s_u) * r_s
    b2c = (s_u * (1.0 / KTAB)) * r_s
    lo = 1e-6 * r_s
    hi = r_s

    def _write(c, buf, sem):
        return pltpu.make_async_copy(
            buf, out_hbm.at[pl.ds(base + c * CH, CH)], sem)

    for c in range(NBUF - 1):
        _gather(probs_hbm, idx_v, c, bufs[c], gsem[c]).start()

    def step(k, carry):
        for j in range(NBUF):
            c = NBUF * k + j
            _gather(probs_hbm, idx_v, c, bufs[j], gsem[j]).wait()

            def make_row(r, j=j):
                def body(jj, carry):
                    for q in range(UNROLL):
                        col = (jj * UNROLL + q) * L
                        v = bufs[j][r, pl.ds(col, L)]
                        bufs[j][r, pl.ds(col, L)] = jnp.minimum(
                            jnp.maximum(a2 * v + b2c, lo), hi)
                    return carry
                return body

            for r in range(CH):
                lax.fori_loop(0, D // (L * UNROLL), make_row(r), 0)

            _write(c, bufs[j], wsem[j]).start()

            nc = c + NBUF - 1
            nj = (j + NBUF - 1) % NBUF

            @pl.when(nc < NCHUNK)
            def _():
                @pl.when(c >= 1)
                def _():
                    _write(c - 1, bufs[nj], wsem[nj]).wait()
                _gather(probs_hbm, idx_v, nc, bufs[nj], gsem[nj]).start()
        return carry

    lax.fori_loop(0, NCHUNK // NBUF, step, 0)

    for c in range(NCHUNK - NBUF, NCHUNK):
        _write(c, bufs[c % NBUF], wsem[c % NBUF]).wait()


def kernel(probs, x, u, t_now, t_next):
    x3 = x.astype(jnp.int32).reshape(NW, BPW // CH, CH)
    part = _sum_kernel(probs, x3, u)
    return _scale_kernel(probs, x3, u, part)
